# Initial kernel scaffold; baseline (speedup 1.0000x reference)
import jax
import jax.numpy as jnp
from jax.experimental import pallas as pl


def kernel(messages, dst, num_nodes):
    raise NotImplementedError("write your pallas kernel here")


# speedup vs baseline: 2.1897x; 2.1897x over previous
import functools

import jax
import jax.numpy as jnp
from jax import lax
from jax.experimental import pallas as pl
from jax.experimental.pallas import tpu as pltpu
from jax.experimental.pallas import tpu_sc as plsc

N = 10000
E = 320000
D = 128
L = 16
NC, NS = 2, 16
NW = NC * NS
NPT = 160
VT = 2 * NW
NPAD = VT * NPT
K = 6144
G = 256
CHUNK = 4000


def _sc_body(msg_hbm, dst_hbm, sum_hbm, sq_hbm, max_hbm, cnt_hbm,
             dstbuf, idx0, off0, idx1, off1, rowbuf,
             acc_s, acc_q, acc_m, acc_c, sem):
    c = lax.axis_index("c")
    s = lax.axis_index("s")
    w = s * NC + c
    lo0 = (2 * w) * NPT
    lo1 = lo0 + NPT
    iota = lax.iota(jnp.int32, L)
    zf = jnp.zeros((L,), jnp.float32)
    ninf = jnp.full((L,), -jnp.inf, jnp.float32)

    def initbuf(j, carry):
        v = j * L + iota
        idx0[pl.ds(j * L, L)] = v
        idx1[pl.ds(j * L, L)] = v
        off0[pl.ds(j * L, L)] = jnp.zeros((L,), jnp.int32)
        off1[pl.ds(j * L, L)] = jnp.zeros((L,), jnp.int32)
        return carry
    lax.fori_loop(0, K // L, initbuf, 0)

    def scan_chunk(cb, carry):
        pltpu.sync_copy(dst_hbm.at[pl.ds(cb * CHUNK, CHUNK)], dstbuf)

        def scan_vec(j, carry):
            cnt0, cnt1 = carry
            v = dstbuf[pl.ds(j * L, L)]
            e = (cb * CHUNK + j * L) + iota
            o0 = v - lo0
            m0 = (v >= lo0) & (o0 < NPT)
            o1 = v - lo1
            m1 = (v >= lo1) & (o1 < NPT)
            p0 = plsc.cumsum(m0.astype(jnp.int32))
            p1 = plsc.cumsum(m1.astype(jnp.int32))
            pos0 = cnt0 + p0 - 1
            pos1 = cnt1 + p1 - 1
            plsc.store_scatter(idx0, [pos0], e, mask=m0)
            plsc.store_scatter(off0, [pos0], o0, mask=m0)
            plsc.store_scatter(idx1, [pos1], e, mask=m1)
            plsc.store_scatter(off1, [pos1], o1, mask=m1)
            cnt0 = jnp.minimum(cnt0 + jnp.sum(m0.astype(jnp.int32)), K - L)
            cnt1 = jnp.minimum(cnt1 + jnp.sum(m1.astype(jnp.int32)), K - L)
            return cnt0, cnt1

        return lax.fori_loop(0, CHUNK // L, scan_vec, carry)

    cnt0, cnt1 = lax.fori_loop(0, E // CHUNK, scan_chunk,
                               (jnp.int32(0), jnp.int32(0)))

    for r, (idxb, offb, cnt) in enumerate(((idx0, off0, cnt0),
                                           (idx1, off1, cnt1))):
        def zacc(j, carry):
            acc_s[pl.ds(j * L, L)] = zf
            acc_q[pl.ds(j * L, L)] = zf
            acc_m[pl.ds(j * L, L)] = ninf
            return carry
        lax.fori_loop(0, NPT * D // L, zacc, 0)

        def zcnt(j, carry):
            acc_c[pl.ds(j * L, L)] = zf
            return carry
        lax.fori_loop(0, NPT // L, zcnt, 0)

        nb = (cnt + (G - 1)) // G

        def batch(b, carry):
            pltpu.async_copy(
                msg_hbm.at[idxb.at[pl.ds(b * G, G)]], rowbuf, sem).wait()

            def edge(e, carry):
                gi = b * G + e
                valid = gi < cnt
                m = jnp.broadcast_to(valid, (L,))
                offv = plsc.load_gather(offb, [jnp.full((L,), gi, jnp.int32)])
                base = offv * D
                for g in range(D // L):
                    row = rowbuf[e, pl.ds(g * L, L)]
                    addr = base + (g * L) + iota
                    plsc.addupdate_scatter(acc_s, [addr], row, mask=m)
                    plsc.addupdate_scatter(acc_q, [addr], row * row, mask=m)
                    cur = plsc.load_gather(acc_m, [addr], mask=m)
                    plsc.store_scatter(acc_m, [addr], jnp.maximum(cur, row),
                                       mask=m)
                plsc.addupdate_scatter(acc_c, [offv],
                                       jnp.full((L,), 1.0, jnp.float32),
                                       mask=m & (iota == 0))
                return carry

            lax.fori_loop(0, G, edge, 0)
            return carry

        lax.fori_loop(0, nb, batch, 0)

        vbase = (2 * w + r) * NPT
        pltpu.sync_copy(acc_s, sum_hbm.at[pl.ds(vbase * D, NPT * D)])
        pltpu.sync_copy(acc_q, sq_hbm.at[pl.ds(vbase * D, NPT * D)])
        pltpu.sync_copy(acc_m, max_hbm.at[pl.ds(vbase * D, NPT * D)])
        pltpu.sync_copy(acc_c, cnt_hbm.at[pl.ds(vbase, NPT)])


_sc_aggregate = functools.partial(
    pl.kernel,
    out_type=(jax.ShapeDtypeStruct((NPAD * D,), jnp.float32),
              jax.ShapeDtypeStruct((NPAD * D,), jnp.float32),
              jax.ShapeDtypeStruct((NPAD * D,), jnp.float32),
              jax.ShapeDtypeStruct((NPAD,), jnp.float32)),
    mesh=plsc.VectorSubcoreMesh(core_axis_name="c", subcore_axis_name="s",
                                num_cores=NC, num_subcores=NS),
    compiler_params=pltpu.CompilerParams(needs_layout_passes=False),
    scratch_types=[
        pltpu.VMEM((CHUNK,), jnp.int32),
        pltpu.VMEM((K,), jnp.int32),
        pltpu.VMEM((K,), jnp.int32),
        pltpu.VMEM((K,), jnp.int32),
        pltpu.VMEM((K,), jnp.int32),
        pltpu.VMEM((G, D), jnp.float32),
        pltpu.VMEM((NPT * D,), jnp.float32),
        pltpu.VMEM((NPT * D,), jnp.float32),
        pltpu.VMEM((NPT * D,), jnp.float32),
        pltpu.VMEM((NPT,), jnp.float32),
        pltpu.SemaphoreType.DMA,
    ],
)(_sc_body)


_ROWS = 256


def _post_body(sum_ref, sq_ref, max_ref, cnt_ref, out_ref):
    sm = sum_ref[...]
    q = sq_ref[...]
    mx = max_ref[...]
    cn = cnt_ref[...]
    deg = jnp.maximum(cn, 1.0)
    rdeg = 1.0 / deg
    mean = sm * rdeg
    var = jnp.maximum(q * rdeg - mean * mean, 0.0)
    std = jnp.sqrt(var + 1e-8)
    mxo = jnp.where(mx == -jnp.inf, 0.0, mx)
    out_ref[...] = jnp.concatenate([sm, mean, mxo, std], axis=-1)


def _postprocess(sums, sqs, maxs, cnts):
    return pl.pallas_call(
        _post_body,
        grid=(NPAD // _ROWS,),
        in_specs=[
            pl.BlockSpec((_ROWS, D), lambda i: (i, 0)),
            pl.BlockSpec((_ROWS, D), lambda i: (i, 0)),
            pl.BlockSpec((_ROWS, D), lambda i: (i, 0)),
            pl.BlockSpec((_ROWS, 1), lambda i: (i, 0)),
        ],
        out_specs=pl.BlockSpec((_ROWS, 4 * D), lambda i: (i, 0)),
        out_shape=jax.ShapeDtypeStruct((NPAD, 4 * D), jnp.float32),
    )(sums, sqs, maxs, cnts)


def kernel(messages, dst, num_nodes):
    del num_nodes
    sum_f, sq_f, max_f, cnt_f = _sc_aggregate(messages, dst)
    out = _postprocess(sum_f.reshape(NPAD, D), sq_f.reshape(NPAD, D),
                       max_f.reshape(NPAD, D), cnt_f.reshape(NPAD, 1))
    return out[:N]


# speedup vs baseline: 3.5438x; 1.6184x over previous
import jax
import jax.numpy as jnp
from jax import lax
from jax.experimental import pallas as pl
from jax.experimental.pallas import tpu as pltpu
from jax.experimental.pallas import tpu_sc as plsc

N = 10000
E = 320000
D = 128
L = 16
NC, NS = 2, 16
NW = NC * NS
NPT = 160
TR = NPT
NPTA = NPT + 1
VT = 2 * NW
NPAD = VT * NPT
K = 6400
G = 128
CBANK = 176
CHUNK = 4000


def _sc_body(msg_hbm, dst_hbm, sum_hbm, sq_hbm, max_hbm, cnt_hbm,
             dstbuf, idx0, off0, idx1, off1, rbA, rbB,
             acc_s, acc_q, acc_m, acc_c2, acc_cr, semA, semB):
    c = lax.axis_index("c")
    s = lax.axis_index("s")
    w = s * NC + c
    lo0 = w * (2 * NPT)
    iota = lax.iota(jnp.int32, L)
    zf = jnp.zeros((L,), jnp.float32)
    ninf = jnp.full((L,), -jnp.inf, jnp.float32)
    ones_f = jnp.full((L,), 1.0, jnp.float32)
    trash = jnp.full((L,), TR, jnp.int32)

    def initbuf(j, carry):
        v = j * L + iota
        idx0[pl.ds(j * L, L)] = v
        idx1[pl.ds(j * L, L)] = v
        off0[pl.ds(j * L, L)] = trash
        off1[pl.ds(j * L, L)] = trash
        return carry
    lax.fori_loop(0, K // L, initbuf, 0)

    def scan_chunk(cb, carry):
        pltpu.sync_copy(dst_hbm.at[pl.ds(cb * CHUNK, CHUNK)], dstbuf)

        def scan_vec(j, carry):
            cnt0, cnt1 = carry
            v = dstbuf[pl.ds(j * L, L)]
            e = (cb * CHUNK + j * L) + iota
            o = v - lo0
            m0 = (o >= 0) & (o < NPT)
            o1 = o - NPT
            m1 = (o1 >= 0) & (o1 < NPT)
            p0 = plsc.cumsum(m0.astype(jnp.int32))
            p1 = plsc.cumsum(m1.astype(jnp.int32))
            pos0 = cnt0 + p0 - 1
            pos1 = cnt1 + p1 - 1
            plsc.store_scatter(idx0, [pos0], e, mask=m0)
            plsc.store_scatter(off0, [pos0], o, mask=m0)
            plsc.store_scatter(idx1, [pos1], e, mask=m1)
            plsc.store_scatter(off1, [pos1], o1, mask=m1)
            cnt0 = jnp.minimum(cnt0 + p0[L - 1], K - L)
            cnt1 = jnp.minimum(cnt1 + p1[L - 1], K - L)
            return cnt0, cnt1

        return lax.fori_loop(0, CHUNK // L, scan_vec, carry)

    cnt0, cnt1 = lax.fori_loop(0, E // CHUNK, scan_chunk,
                               (jnp.int32(0), jnp.int32(0)))

    for r, (idxb, offb, cnt) in enumerate(((idx0, off0, cnt0),
                                           (idx1, off1, cnt1))):
        def zacc(j, carry):
            acc_s[pl.ds(j * L, L)] = zf
            acc_q[pl.ds(j * L, L)] = zf
            acc_m[pl.ds(j * L, L)] = ninf
            return carry
        lax.fori_loop(0, NPTA * D // L, zacc, 0)

        def zcnt(j, carry):
            acc_c2[pl.ds(j * L, L)] = zf
            return carry
        lax.fori_loop(0, L * CBANK // L, zcnt, 0)

        nb = jnp.maximum((cnt + (G - 1)) // G, 1)
        bmax = nb - 1

        def fire(b, buf, sem):
            pltpu.async_copy(msg_hbm.at[idxb.at[pl.ds(b * G, G)]], buf, sem)

        def drain(buf, sem):
            pltpu.make_async_copy(
                msg_hbm.at[idxb.at[pl.ds(0, G)]], buf, sem).wait()

        def process(b, buf):
            def group(q, carry):
                gbase = b * G + q * L
                offv = offb[pl.ds(gbase, L)]
                plsc.addupdate_scatter(acc_c2, [iota * CBANK + offv], ones_f)
                for e in range(L):
                    abase = offv[e] * D
                    rrow = q * L + e
                    for g in range(D // L):
                        row = buf[rrow, pl.ds(g * L, L)]
                        sl = pl.ds(abase + g * L, L)
                        plsc.addupdate(acc_s.at[sl], row)
                        plsc.addupdate(acc_q.at[sl], row * row)
                        acc_m[sl] = jnp.maximum(acc_m[sl], row)
                return carry
            lax.fori_loop(0, G // L, group, 0)

        fire(0, rbA, semA)

        def pair(b2, carry):
            bA = 2 * b2
            bB = bA + 1
            drain(rbA, semA)
            fire(jnp.minimum(bB, bmax), rbB, semB)
            process(bA, rbA)
            drain(rbB, semB)
            fire(jnp.minimum(bA + 2, bmax), rbA, semA)

            @pl.when(bB < nb)
            def _():
                process(bB, rbB)
            return carry

        lax.fori_loop(0, (nb + 1) // 2, pair, 0)
        drain(rbA, semA)

        def credu(ci, carry):
            acc = zf
            for l in range(L):
                acc = acc + acc_c2[pl.ds(l * CBANK + ci * L, L)]
            acc_cr[pl.ds(ci * L, L)] = acc
            return carry
        lax.fori_loop(0, NPT // L, credu, 0)

        vbase = (2 * w + r) * NPT
        pltpu.sync_copy(acc_s.at[pl.ds(0, NPT * D)],
                        sum_hbm.at[pl.ds(vbase * D, NPT * D)])
        pltpu.sync_copy(acc_q.at[pl.ds(0, NPT * D)],
                        sq_hbm.at[pl.ds(vbase * D, NPT * D)])
        pltpu.sync_copy(acc_m.at[pl.ds(0, NPT * D)],
                        max_hbm.at[pl.ds(vbase * D, NPT * D)])
        pltpu.sync_copy(acc_cr, cnt_hbm.at[pl.ds(vbase, NPT)])


_sc_aggregate = pl.kernel(
    _sc_body,
    out_type=(jax.ShapeDtypeStruct((NPAD * D,), jnp.float32),
              jax.ShapeDtypeStruct((NPAD * D,), jnp.float32),
              jax.ShapeDtypeStruct((NPAD * D,), jnp.float32),
              jax.ShapeDtypeStruct((NPAD,), jnp.float32)),
    mesh=plsc.VectorSubcoreMesh(core_axis_name="c", subcore_axis_name="s",
                                num_cores=NC, num_subcores=NS),
    compiler_params=pltpu.CompilerParams(needs_layout_passes=False),
    scratch_types=[
        pltpu.VMEM((CHUNK,), jnp.int32),
        pltpu.VMEM((K,), jnp.int32),
        pltpu.VMEM((K,), jnp.int32),
        pltpu.VMEM((K,), jnp.int32),
        pltpu.VMEM((K,), jnp.int32),
        pltpu.VMEM((G, D), jnp.float32),
        pltpu.VMEM((G, D), jnp.float32),
        pltpu.VMEM((NPTA * D,), jnp.float32),
        pltpu.VMEM((NPTA * D,), jnp.float32),
        pltpu.VMEM((NPTA * D,), jnp.float32),
        pltpu.VMEM((L * CBANK,), jnp.float32),
        pltpu.VMEM((NPT,), jnp.float32),
        pltpu.SemaphoreType.DMA,
        pltpu.SemaphoreType.DMA,
    ],
)


_ROWS = 256


def _post_body(sum_ref, sq_ref, max_ref, cnt_ref, out_ref):
    sm = sum_ref[...]
    q = sq_ref[...]
    mx = max_ref[...]
    cn = cnt_ref[...]
    deg = jnp.maximum(cn, 1.0)
    rdeg = 1.0 / deg
    mean = sm * rdeg
    var = jnp.maximum(q * rdeg - mean * mean, 0.0)
    std = jnp.sqrt(var + 1e-8)
    mxo = jnp.where(mx == -jnp.inf, 0.0, mx)
    out_ref[...] = jnp.concatenate([sm, mean, mxo, std], axis=-1)


def _postprocess(sums, sqs, maxs, cnts):
    return pl.pallas_call(
        _post_body,
        grid=(NPAD // _ROWS,),
        in_specs=[
            pl.BlockSpec((_ROWS, D), lambda i: (i, 0)),
            pl.BlockSpec((_ROWS, D), lambda i: (i, 0)),
            pl.BlockSpec((_ROWS, D), lambda i: (i, 0)),
            pl.BlockSpec((_ROWS, 1), lambda i: (i, 0)),
        ],
        out_specs=pl.BlockSpec((_ROWS, 4 * D), lambda i: (i, 0)),
        out_shape=jax.ShapeDtypeStruct((NPAD, 4 * D), jnp.float32),
    )(sums, sqs, maxs, cnts)


def kernel(messages, dst, num_nodes):
    del num_nodes
    sum_f, sq_f, max_f, cnt_f = _sc_aggregate(messages, dst)
    out = _postprocess(sum_f.reshape(NPAD, D), sq_f.reshape(NPAD, D),
                       max_f.reshape(NPAD, D), cnt_f.reshape(NPAD, 1))
    return out[:N]
